# baseline (speedup 1.0000x reference)
import functools

import jax
import jax.numpy as jnp
from jax import lax
from jax.experimental import pallas as pl
from jax.experimental.pallas import tpu as pltpu
from jax.experimental.pallas import tpu_sc as plsc

_E = 16
_D = 2048
_SCE = 4
_TCE = _E - _SCE
_BD = 1024
_NB = _D // _BD

_NC = 2
_NS = 16
_ROWS = _D // (_NC * _NS)
_CHUNK = 16
_GW = 8
_STRIP = _D // _NS


def _tc_body(x_ref, wg_ref, bg_ref, wj_ref, bj_ref, z_ref, we_ref, be_ref,
             out_ref, gate_ref, xg_ref):
    e = pl.program_id(0)
    j = pl.program_id(1)

    @pl.when((e == 0) & (j == 0))
    def _router():
        xv = x_ref[...]
        logits = jnp.dot(xv, wg_ref[...],
                         preferred_element_type=jnp.float32) + bg_ref[...]
        pre = jnp.dot(xv, wj_ref[...],
                      preferred_element_type=jnp.float32) + bj_ref[...]
        scales = jax.nn.softplus(pre)
        t = logits + scales * z_ref[...]
        iota = lax.broadcasted_iota(jnp.int32, (1, _E), 1)
        m1 = jnp.max(t)
        i1 = jnp.min(jnp.where(t == m1, iota, _E))
        masked = jnp.where(iota == i1, -jnp.inf, t)
        m2 = jnp.max(masked)
        i2 = jnp.min(jnp.where(masked == m2, iota, _E))
        sel = (iota == i1) | (iota == i2)
        sparse = jnp.where(sel, t, 0.0)
        g = jnp.exp(sparse - jnp.max(sparse))
        gate = g / jnp.sum(g)
        gate_ref[...] = gate
        out_ref[...] = jnp.dot(gate, be_ref[...],
                               preferred_element_type=jnp.float32)
        xg_ref[...] = lax.dot_general(
            gate, xv, dimension_numbers=(((0,), (0,)), ((), ())),
            preferred_element_type=jnp.float32)

    col = pl.multiple_of(j * _BD, _BD)
    xg_row = xg_ref[pl.ds(e, 1), pl.ds(col, _BD)]
    out_ref[...] += jnp.dot(xg_row, we_ref[0],
                            preferred_element_type=jnp.float32)


def _sc_body(x_hbm, we_hbm, out_hbm, x_v, bufs_v, acc_v, red_v, strip_v,
             shared_v, sems):
    c = lax.axis_index("c")
    s = lax.axis_index("s")
    d_base = c * (_NS * _ROWS) + s * _ROWS
    nch = _ROWS // _CHUNK

    pltpu.sync_copy(x_hbm.at[0, pl.ds(d_base, _ROWS)], x_v)

    for idx in range(_SCE):
        e_abs = _TCE + idx

        def _copy(t, slot, _e=e_abs):
            return pltpu.make_async_copy(
                we_hbm.at[_e, pl.ds(d_base + t * _CHUNK, _CHUNK), :],
                bufs_v.at[slot], sems.at[slot])

        def zero_body(g, _):
            acc_v[pl.ds(pl.multiple_of(g * 16, 16), 16)] = (
                jnp.zeros((16,), jnp.float32))
            return 0
        lax.fori_loop(0, _D // 16, zero_body, 0)
        _copy(0, 0).start()
        _copy(1, 1).start()

        def chunk_body(t, _):
            slot = lax.rem(t, 2)
            _copy(t, slot).wait()
            xs16 = x_v[pl.ds(pl.multiple_of(t * _CHUNK, _CHUNK), _CHUNK)]

            def group_body(g, _):
                f0 = pl.multiple_of(g * 16 * _GW, 16 * _GW)
                accs = tuple(acc_v[pl.ds(f0 + 16 * k, 16)]
                             for k in range(_GW))
                for l in range(_CHUNK):
                    xs = xs16[l]
                    accs = tuple(
                        a + xs * bufs_v[slot, l, pl.ds(f0 + 16 * k, 16)]
                        for k, a in enumerate(accs))
                for k in range(_GW):
                    acc_v[pl.ds(f0 + 16 * k, 16)] = accs[k]
                return 0

            lax.fori_loop(0, _D // (16 * _GW), group_body, 0)

            @pl.when(t + 2 < nch)
            def _():
                _copy(t + 2, slot).start()
            return 0

        lax.fori_loop(0, nch, chunk_body, 0)

        pltpu.sync_copy(acc_v, shared_v.at[s])
        plsc.subcore_barrier()
        pltpu.sync_copy(shared_v.at[:, pl.ds(s * _STRIP, _STRIP)], red_v)
        for k in range(_STRIP // 16):
            t = red_v[0, pl.ds(16 * k, 16)]
            for jrow in range(1, _NS):
                t = t + red_v[jrow, pl.ds(16 * k, 16)]
            strip_v[pl.ds(16 * k, 16)] = t
        pltpu.sync_copy(strip_v,
                        out_hbm.at[c * _SCE + idx, pl.ds(s * _STRIP, _STRIP)])
        plsc.subcore_barrier()


def _combine_body(outp_ref, gate_ref, rsc_ref, o_ref):
    acc = outp_ref[...]
    for c in range(_NC):
        for i in range(_SCE):
            w = gate_ref[0, _TCE + i]
            acc += w * rsc_ref[pl.ds(c * _SCE + i, 1), :]
    o_ref[...] = acc


@jax.jit
def kernel(x, Wg, bg, Wj, bj, We, be, z):
    x2 = x.reshape(1, _D)
    bg2 = bg.reshape(1, _E)
    bj2 = bj.reshape(1, _E)
    z2 = z.reshape(1, _E)

    outp, gate = pl.pallas_call(
        _tc_body,
        grid=(_TCE, _NB),
        in_specs=[
            pl.BlockSpec((1, _D), lambda e, j: (0, 0)),
            pl.BlockSpec((_D, _E), lambda e, j: (0, 0)),
            pl.BlockSpec((1, _E), lambda e, j: (0, 0)),
            pl.BlockSpec((_D, _E), lambda e, j: (0, 0)),
            pl.BlockSpec((1, _E), lambda e, j: (0, 0)),
            pl.BlockSpec((1, _E), lambda e, j: (0, 0)),
            pl.BlockSpec((1, _BD, _D), lambda e, j: (e, j, 0)),
            pl.BlockSpec((_E, _D), lambda e, j: (0, 0)),
        ],
        out_specs=[
            pl.BlockSpec((1, _D), lambda e, j: (0, 0)),
            pl.BlockSpec((1, _E), lambda e, j: (0, 0)),
        ],
        out_shape=[
            jax.ShapeDtypeStruct((1, _D), jnp.float32),
            jax.ShapeDtypeStruct((1, _E), jnp.float32),
        ],
        scratch_shapes=[pltpu.VMEM((_E, _D), jnp.float32)],
    )(x2, Wg, bg2, Wj, bj2, z2, We, be)

    sc_kernel = functools.partial(
        pl.kernel,
        mesh=plsc.VectorSubcoreMesh(core_axis_name="c", subcore_axis_name="s"),
        out_type=jax.ShapeDtypeStruct((_NC * _SCE, _D), jnp.float32),
        scratch_types=[
            pltpu.VMEM((_ROWS,), jnp.float32),
            pltpu.VMEM((2, _CHUNK, _D), jnp.float32),
            pltpu.VMEM((_D,), jnp.float32),
            pltpu.VMEM((_NS, _STRIP), jnp.float32),
            pltpu.VMEM((_STRIP,), jnp.float32),
            pltpu.VMEM_SHARED((_NS, _D), jnp.float32),
            pltpu.SemaphoreType.DMA((2,)),
        ],
    )(_sc_body)
    r_sc2 = sc_kernel(x2, We)

    out = pl.pallas_call(
        _combine_body,
        in_specs=[
            pl.BlockSpec((1, _D), lambda: (0, 0)),
            pl.BlockSpec(memory_space=pltpu.SMEM),
            pl.BlockSpec((_NC * _SCE, _D), lambda: (0, 0)),
        ],
        out_specs=pl.BlockSpec((1, _D), lambda: (0, 0)),
        out_shape=jax.ShapeDtypeStruct((1, _D), jnp.float32),
    )(outp, gate, r_sc2)
    return out.reshape(_D)


# speedup vs baseline: 1.2062x; 1.2062x over previous
import jax
import jax.numpy as jnp
from jax import lax
from jax.experimental import pallas as pl
from jax.experimental.pallas import tpu as pltpu

_E = 16
_D = 2048
_CR = 512
_NCH = _E * _D // _CR
_PER_E = _D // _CR
_NBUF = 4


def _moe_body(x_ref, wg_ref, bg_ref, wj_ref, bj_ref, z_ref, we_any, be_ref,
              out_ref, xg_ref, bufs, sems):
    def _copy(t, slot):
        return pltpu.make_async_copy(we_any.at[t], bufs.at[slot],
                                     sems.at[slot])

    for t in range(_NBUF):
        _copy(t, t).start()

    xv = x_ref[...]
    logits = jnp.dot(xv, wg_ref[...],
                     preferred_element_type=jnp.float32) + bg_ref[...]
    pre = jnp.dot(xv, wj_ref[...],
                  preferred_element_type=jnp.float32) + bj_ref[...]
    scales = jax.nn.softplus(pre)
    tj = logits + scales * z_ref[...]
    iota = lax.broadcasted_iota(jnp.int32, (1, _E), 1)
    m1 = jnp.max(tj)
    i1 = jnp.min(jnp.where(tj == m1, iota, _E))
    masked = jnp.where(iota == i1, -jnp.inf, tj)
    m2 = jnp.max(masked)
    i2 = jnp.min(jnp.where(masked == m2, iota, _E))
    sel = (iota == i1) | (iota == i2)
    sparse = jnp.where(sel, tj, 0.0)
    g = jnp.exp(sparse - jnp.max(sparse))
    gate = g / jnp.sum(g)
    out_ref[...] = jnp.dot(gate, be_ref[...],
                           preferred_element_type=jnp.float32)
    xg_ref[...] = lax.dot_general(
        gate, xv, dimension_numbers=(((0,), (0,)), ((), ())),
        preferred_element_type=jnp.float32)

    def chunk_body(t, _):
        slot = lax.rem(t, _NBUF)
        _copy(t, slot).wait()
        e = t // _PER_E
        col = pl.multiple_of(lax.rem(t, _PER_E) * _CR, _CR)
        xg_row = xg_ref[pl.ds(e, 1), pl.ds(col, _CR)]
        out_ref[...] += jnp.dot(xg_row, bufs[slot],
                                preferred_element_type=jnp.float32)

        @pl.when(t + _NBUF < _NCH)
        def _():
            _copy(t + _NBUF, slot).start()
        return 0

    lax.fori_loop(0, _NCH, chunk_body, 0)


@jax.jit
def kernel(x, Wg, bg, Wj, bj, We, be, z):
    x2 = x.reshape(1, _D)
    bg2 = bg.reshape(1, _E)
    bj2 = bj.reshape(1, _E)
    z2 = z.reshape(1, _E)
    We3 = We.reshape(_NCH, _CR, _D)

    out = pl.pallas_call(
        _moe_body,
        grid=(1,),
        in_specs=[
            pl.BlockSpec((1, _D), lambda i: (0, 0)),
            pl.BlockSpec((_D, _E), lambda i: (0, 0)),
            pl.BlockSpec((1, _E), lambda i: (0, 0)),
            pl.BlockSpec((_D, _E), lambda i: (0, 0)),
            pl.BlockSpec((1, _E), lambda i: (0, 0)),
            pl.BlockSpec((1, _E), lambda i: (0, 0)),
            pl.BlockSpec(memory_space=pltpu.MemorySpace.HBM),
            pl.BlockSpec((_E, _D), lambda i: (0, 0)),
        ],
        out_specs=pl.BlockSpec((1, _D), lambda i: (0, 0)),
        out_shape=jax.ShapeDtypeStruct((1, _D), jnp.float32),
        scratch_shapes=[
            pltpu.VMEM((_E, _D), jnp.float32),
            pltpu.VMEM((_NBUF, _CR, _D), jnp.float32),
            pltpu.SemaphoreType.DMA((_NBUF,)),
        ],
    )(x2, Wg, bg2, Wj, bj2, z2, We3, be)
    return out.reshape(_D)


# speedup vs baseline: 1.2149x; 1.0072x over previous
import jax
import jax.numpy as jnp
from jax import lax
from jax.experimental import pallas as pl
from jax.experimental.pallas import tpu as pltpu

_E = 16
_D = 2048
_BD = 1024
_NB = _D // _BD


def _moe_body(x_ref, wg_ref, bg_ref, wj_ref, bj_ref, z_ref, we_ref, be_ref,
              out_ref, xg_ref):
    e = pl.program_id(0)
    j = pl.program_id(1)

    @pl.when((e == 0) & (j == 0))
    def _router():
        xv = x_ref[...]
        logits = jnp.dot(xv, wg_ref[...],
                         preferred_element_type=jnp.float32) + bg_ref[...]
        pre = jnp.dot(xv, wj_ref[...],
                      preferred_element_type=jnp.float32) + bj_ref[...]
        scales = jax.nn.softplus(pre)
        t = logits + scales * z_ref[...]
        iota = lax.broadcasted_iota(jnp.int32, (1, _E), 1)
        m1 = jnp.max(t)
        i1 = jnp.min(jnp.where(t == m1, iota, _E))
        masked = jnp.where(iota == i1, -jnp.inf, t)
        m2 = jnp.max(masked)
        i2 = jnp.min(jnp.where(masked == m2, iota, _E))
        sel = (iota == i1) | (iota == i2)
        sparse = jnp.where(sel, t, 0.0)
        g = jnp.exp(sparse - jnp.max(sparse))
        gate = g / jnp.sum(g)
        out_ref[...] = jnp.dot(gate, be_ref[...],
                               preferred_element_type=jnp.float32)
        xg_ref[...] = lax.dot_general(
            gate, xv, dimension_numbers=(((0,), (0,)), ((), ())),
            preferred_element_type=jnp.float32)

    col = pl.multiple_of(j * _BD, _BD)
    xg_row = xg_ref[pl.ds(e, 1), pl.ds(col, _BD)]
    out_ref[...] += jnp.dot(xg_row, we_ref[0],
                            preferred_element_type=jnp.float32)


@jax.jit
def kernel(x, Wg, bg, Wj, bj, We, be, z):
    x2 = x.reshape(1, _D)
    bg2 = bg.reshape(1, _E)
    bj2 = bj.reshape(1, _E)
    z2 = z.reshape(1, _E)

    out = pl.pallas_call(
        _moe_body,
        grid=(_E, _NB),
        in_specs=[
            pl.BlockSpec((1, _D), lambda e, j: (0, 0)),
            pl.BlockSpec((_D, _E), lambda e, j: (0, 0)),
            pl.BlockSpec((1, _E), lambda e, j: (0, 0)),
            pl.BlockSpec((_D, _E), lambda e, j: (0, 0)),
            pl.BlockSpec((1, _E), lambda e, j: (0, 0)),
            pl.BlockSpec((1, _E), lambda e, j: (0, 0)),
            pl.BlockSpec((1, _BD, _D), lambda e, j: (e, j, 0)),
            pl.BlockSpec((_E, _D), lambda e, j: (0, 0)),
        ],
        out_specs=pl.BlockSpec((1, _D), lambda e, j: (0, 0)),
        out_shape=jax.ShapeDtypeStruct((1, _D), jnp.float32),
        scratch_shapes=[pltpu.VMEM((_E, _D), jnp.float32)],
    )(x2, Wg, bg2, Wj, bj2, z2, We, be)
    return out.reshape(_D)
